# baseline (speedup 1.0000x reference)
import functools

import jax
import jax.numpy as jnp
from jax import lax
from jax.experimental import pallas as pl
from jax.experimental.pallas import tpu as pltpu
from jax.experimental.pallas import tpu_sc as plsc

F = 9
V = 100000
D = 64
T = 4096 * 50
NC, NS = 2, 16
NW = NC * NS
TPW = T // NW
C = 64
NCHUNK = TPW // C
L = 16


def _proj_body(t_ref, w_ref, bb_ref, o_ref):
    prj = jax.lax.dot_general(
        t_ref[0], w_ref[0],
        dimension_numbers=(((0,), (0,)), ((), ())),
        preferred_element_type=jnp.float32,
    ) + bb_ref[0]
    o_ref[0] = jnp.pad(prj, ((0, 0), (0, D)))


_VB = 12800


_project = pl.pallas_call(
    _proj_body,
    grid=(F, (V + _VB - 1) // _VB),
    in_specs=[
        pl.BlockSpec((1, D, _VB), lambda i, v: (i, 0, v)),
        pl.BlockSpec((1, D, D), lambda i, v: (i, 0, 0)),
        pl.BlockSpec((1, 1, D), lambda i, v: (i, 0, 0)),
    ],
    out_specs=pl.BlockSpec((1, _VB, 2 * D), lambda i, v: (i, v, 0)),
    out_shape=jax.ShapeDtypeStruct((F, V, 2 * D), jnp.float32),
)


BLK = 128
BLOCKS = TPW // BLK


@functools.partial(
    pl.kernel,
    out_type=jax.ShapeDtypeStruct((T // 4096, D // 8, 4096 // 128, 8, 128),
                                  jnp.float32),
    mesh=plsc.VectorSubcoreMesh(core_axis_name="c", subcore_axis_name="s"),
    compiler_params=pltpu.CompilerParams(use_tc_tiling_on_sc=False,
                                         needs_layout_passes=False),
    scratch_types=[
        pltpu.VMEM((2, F, BLK), jnp.int32),
        pltpu.VMEM((2, F, C), jnp.int32),
        pltpu.VMEM((2, F, C, D), jnp.float32),
        pltpu.VMEM((2, D // 8, 8, C), jnp.float32),
        pltpu.SemaphoreType.DMA,
        pltpu.SemaphoreType.DMA,
        pltpu.SemaphoreType.DMA,
        pltpu.SemaphoreType.DMA,
        pltpu.SemaphoreType.DMA,
        pltpu.SemaphoreType.DMA,
    ],
)
def _gather_sum(p_hbm, idx_hbm, out_hbm, stg_v, fidx_v, rows_v, out_v,
                s_stg0, s_stg1, s_gat0, s_gat1, s_out0, s_out1):
    stg_sems = (s_stg0, s_stg1)
    gat_sems = (s_gat0, s_gat1)
    out_sems = (s_out0, s_out1)

    wid = lax.axis_index("s") * NC + lax.axis_index("c")
    tok0 = wid * TPW

    def stg_copy(blk, sp):
        off = pl.multiple_of(tok0 + blk * BLK, BLK)
        return pltpu.make_async_copy(
            idx_hbm.at[:, pl.ds(off, BLK)],
            stg_v.at[sp],
            stg_sems[sp],
        )

    def gat_copy(par, i):
        return pltpu.make_async_copy(
            p_hbm.at[fidx_v.at[par, i]],
            rows_v.at[par, i],
            gat_sems[par],
        )

    def out_copy(c, par):
        t0c = tok0 + c * C
        s = lax.div(t0c, 4096)
        boff = lax.rem(t0c, 4096)
        bt = lax.div(boff, 128)
        bh = lax.rem(boff, 128)
        return pltpu.make_async_copy(
            out_v.at[par],
            out_hbm.at[s, :, bt, :, pl.ds(bh, C)],
            out_sems[par],
        )

    def prep_and_launch(par, sp, col0):
        for i in range(F):
            off = jnp.full((L,), 2 * i * V, jnp.int32)
            for q in range(C // L):
                src = stg_v[sp, i, pl.ds(col0 + q * L, L)]
                flat = (src + src) + off if i else src + src
                fidx_v[par, i, pl.ds(q * L, L)] = flat
        for i in range(F):
            gat_copy(par, i).start()

    def reduce_chunk(par):
        iota = lax.iota(jnp.int32, L)

        def body(d, carry):
            dt = lax.div(d, 8)
            d8 = lax.rem(d, 8)
            cvec = jnp.zeros((L,), jnp.int32) + d
            for bs in range(C // L):
                rvec = iota + jnp.int32(bs * L)
                acc = plsc.load_gather(rows_v.at[par, 0], [rvec, cvec])
                for i in range(1, F):
                    acc = acc + plsc.load_gather(rows_v.at[par, i],
                                                 [rvec, cvec])
                out_v[par, dt, d8, pl.ds(bs * L, L)] = acc
            return carry
        lax.fori_loop(0, D, body, 0)

    stg_copy(0, 0).start()
    stg_copy(0, 0).wait()
    prep_and_launch(0, 0, 0)
    stg_copy(1, 1).start()

    def body(go, carry):
        for jb in range(2):
            for jc in range(2):
                c = 4 * go + 2 * jb + jc
                p = jc
                for i in range(F):
                    gat_copy(p, i).wait()

                @pl.when(c + 1 < NCHUNK)
                def _():
                    if jc == 1:
                        stg_copy(2 * go + jb + 1, 1 - jb).wait()
                    sp_next = jb if jc == 0 else 1 - jb
                    prep_and_launch(1 - p, sp_next, C * (1 - jc))

                if jc == 0:
                    nxt = 2 * go + jb + 2

                    @pl.when(nxt < BLOCKS)
                    def _():
                        stg_copy(nxt, jb).start()

                @pl.when(c >= 2)
                def _():
                    out_copy(c - 2, p).wait()

                reduce_chunk(p)
                out_copy(c, p).start()
        return carry

    lax.fori_loop(0, BLOCKS // 2, body, 0)
    out_copy(NCHUNK - 2, 0).wait()
    out_copy(NCHUNK - 1, 1).wait()


def kernel(feature_indices, tables, W, b):
    B_dim, S_dim = feature_indices.shape[0], feature_indices.shape[1]
    bb = jnp.zeros((F, 1, D), jnp.float32).at[0, 0].set(b)
    proj = _project(tables.transpose(0, 2, 1), W.reshape(F, D, D), bb)
    idx_t = feature_indices.transpose(2, 1, 0).reshape(F, T)
    out5 = _gather_sum(proj.reshape(2 * F * V, D), idx_t)
    return out5.transpose(2, 4, 0, 1, 3).reshape(B_dim, S_dim, D)


# speedup vs baseline: 4.1512x; 4.1512x over previous
import functools

import jax
import jax.numpy as jnp
from jax import lax
from jax.experimental import pallas as pl
from jax.experimental.pallas import tpu as pltpu
from jax.experimental.pallas import tpu_sc as plsc

F = 9
V = 100000
D = 64
T = 4096 * 50
NC, NS = 2, 16
NW = NC * NS
TPW = T // NW
C = 64
NCHUNK = TPW // C
L = 16


def _proj_body(t_ref, w_ref, bb_ref, o_ref):
    prj = jax.lax.dot_general(
        t_ref[0], w_ref[0],
        dimension_numbers=(((0,), (0,)), ((), ())),
        preferred_element_type=jnp.float32,
    ) + bb_ref[0]
    o_ref[0] = jnp.pad(prj, ((0, 0), (0, D)))


_VB = 12800


_project = pl.pallas_call(
    _proj_body,
    grid=(F, (V + _VB - 1) // _VB),
    in_specs=[
        pl.BlockSpec((1, D, _VB), lambda i, v: (i, 0, v)),
        pl.BlockSpec((1, D, D), lambda i, v: (i, 0, 0)),
        pl.BlockSpec((1, 1, D), lambda i, v: (i, 0, 0)),
    ],
    out_specs=pl.BlockSpec((1, _VB, 2 * D), lambda i, v: (i, v, 0)),
    out_shape=jax.ShapeDtypeStruct((F, V, 2 * D), jnp.float32),
)


BLK = 128
BLOCKS = TPW // BLK


@functools.partial(
    pl.kernel,
    out_type=jax.ShapeDtypeStruct((T, D), jnp.float32),
    mesh=plsc.VectorSubcoreMesh(core_axis_name="c", subcore_axis_name="s"),
    compiler_params=pltpu.CompilerParams(use_tc_tiling_on_sc=False),
    scratch_types=[
        pltpu.VMEM((2, F, BLK), jnp.int32),
        pltpu.VMEM((2, F, C), jnp.int32),
        pltpu.VMEM((2, F, C, D), jnp.float32),
        pltpu.VMEM((2, C, D), jnp.float32),
        pltpu.SemaphoreType.DMA,
        pltpu.SemaphoreType.DMA,
        pltpu.SemaphoreType.DMA,
        pltpu.SemaphoreType.DMA,
        pltpu.SemaphoreType.DMA,
        pltpu.SemaphoreType.DMA,
    ],
)
def _gather_sum(p_hbm, idx_hbm, out_hbm, stg_v, fidx_v, rows_v, out_v,
                s_stg0, s_stg1, s_gat0, s_gat1, s_out0, s_out1):
    stg_sems = (s_stg0, s_stg1)
    gat_sems = (s_gat0, s_gat1)
    out_sems = (s_out0, s_out1)

    wid = lax.axis_index("s") * NC + lax.axis_index("c")
    tok0 = wid * TPW

    def stg_copy(blk, sp):
        off = pl.multiple_of(tok0 + blk * BLK, BLK)
        return pltpu.make_async_copy(
            idx_hbm.at[:, pl.ds(off, BLK)],
            stg_v.at[sp],
            stg_sems[sp],
        )

    def gat_copy(par, i):
        return pltpu.make_async_copy(
            p_hbm.at[fidx_v.at[par, i]],
            rows_v.at[par, i],
            gat_sems[par],
        )

    def out_copy(c, par):
        off = pl.multiple_of(tok0 + c * C, C)
        return pltpu.make_async_copy(
            out_v.at[par],
            out_hbm.at[pl.ds(off, C)],
            out_sems[par],
        )

    def prep_and_launch(par, sp, col0):
        for i in range(F):
            off = jnp.full((L,), 2 * i * V, jnp.int32)
            for q in range(C // L):
                src = stg_v[sp, i, pl.ds(col0 + q * L, L)]
                flat = (src + src) + off if i else src + src
                fidx_v[par, i, pl.ds(q * L, L)] = flat
        for i in range(F):
            gat_copy(par, i).start()

    def reduce_chunk(par):
        def body(g, carry):
            for dt in range(2):
                t = g + jnp.int32(dt * (C // 2))
                for dc in range(D // L):
                    sl = pl.ds(dc * L, L)
                    acc = rows_v[par, 0, t, sl]
                    for i in range(1, F):
                        acc = acc + rows_v[par, i, t, sl]
                    out_v[par, t, sl] = acc
            return carry
        lax.fori_loop(0, C // 2, body, 0)

    stg_copy(0, 0).start()
    stg_copy(0, 0).wait()
    prep_and_launch(0, 0, 0)
    stg_copy(1, 1).start()

    def body(go, carry):
        for jb in range(2):
            for jc in range(2):
                c = 4 * go + 2 * jb + jc
                p = jc
                for i in range(F):
                    gat_copy(p, i).wait()

                @pl.when(c + 1 < NCHUNK)
                def _():
                    if jc == 1:
                        stg_copy(2 * go + jb + 1, 1 - jb).wait()
                    sp_next = jb if jc == 0 else 1 - jb
                    prep_and_launch(1 - p, sp_next, C * (1 - jc))

                if jc == 0:
                    nxt = 2 * go + jb + 2

                    @pl.when(nxt < BLOCKS)
                    def _():
                        stg_copy(nxt, jb).start()

                @pl.when(c >= 2)
                def _():
                    out_copy(c - 2, p).wait()

                reduce_chunk(p)
                out_copy(c, p).start()
        return carry

    lax.fori_loop(0, BLOCKS // 2, body, 0)
    out_copy(NCHUNK - 2, 0).wait()
    out_copy(NCHUNK - 1, 1).wait()


def kernel(feature_indices, tables, W, b):
    B_dim, S_dim = feature_indices.shape[0], feature_indices.shape[1]
    bb = jnp.zeros((F, 1, D), jnp.float32).at[0, 0].set(b)
    proj = _project(tables.transpose(0, 2, 1), W.reshape(F, D, D), bb)
    idx_t = feature_indices.transpose(2, 1, 0).reshape(F, T)
    out_flat = _gather_sum(proj.reshape(2 * F * V, D), idx_t)
    return out_flat.reshape(S_dim, B_dim, D).transpose(1, 0, 2)
